# Initial kernel scaffold; baseline (speedup 1.0000x reference)
import jax
import jax.numpy as jnp
from jax.experimental import pallas as pl


def kernel(x, codebooks):
    raise NotImplementedError("write your pallas kernel here")


# speedup vs baseline: 1.2673x; 1.2673x over previous
import functools

import jax
import jax.numpy as jnp
from jax import lax
from jax.experimental import pallas as pl
from jax.experimental.pallas import tpu as pltpu
from jax.experimental.pallas import tpu_sc as plsc

G = 2
Q = 2
K = 8192
D = 256
M = 2048
KBLK = 1024
NKB = K // KBLK


def _scores(xf, cb):
    xn = jnp.sum(xf * xf, axis=1)
    cn = jnp.sum(cb * cb, axis=1)
    d = lax.dot_general(xf, cb, (((1,), (1,)), ((), ())),
                        preferred_element_type=jnp.float32)
    return (xn[:, None] - 2.0 * d) + cn[None, :]


def _argmin_update(scores, kb, idx_ref, minv):
    lmin = jnp.min(scores, axis=1)
    iota = lax.broadcasted_iota(jnp.int32, scores.shape, 1)
    lidx = jnp.min(jnp.where(scores == lmin[:, None], iota, KBLK), axis=1)
    gidx = (lidx + kb * KBLK)[None, :]
    lmin = lmin[None, :]

    @pl.when(kb == 0)
    def _():
        minv[...] = lmin
        idx_ref[0] = gidx

    @pl.when(kb > 0)
    def _():
        prev = minv[...]
        better = lmin < prev
        idx_ref[0] = jnp.where(better, gidx, idx_ref[0])
        minv[...] = jnp.where(better, lmin, prev)


def _layer1_body(x_ref, cb_ref, idx_ref, minv):
    kb = pl.program_id(1)
    _argmin_update(_scores(x_ref[0], cb_ref[0]), kb, idx_ref, minv)


def _layer2_body(x_ref, xd_ref, cb_ref, idx0_ref, idx1_ref, c0_ref, h_ref,
                 minv):
    kb = pl.program_id(1)
    xf = x_ref[0]
    r = xf - xd_ref[0]
    _argmin_update(_scores(r, cb_ref[0]), kb, idx1_ref, minv)

    idx0 = idx0_ref[0].reshape(M, 1)
    bins = kb * KBLK + lax.broadcasted_iota(jnp.int32, (1, KBLK), 1)
    counts = jnp.sum((idx0 == bins).astype(jnp.float32), axis=0)
    prob = counts * (1.0 / M)
    hblk = jnp.sum((prob * jnp.log(prob + 1e-07))[None, :], axis=1,
                   keepdims=True)

    @pl.when(kb == 0)
    def _():
        h_ref[0] = hblk
        c0_ref[0] = jnp.sum((r * r).reshape(1, M * D), axis=1,
                            keepdims=True) * (1.0 / (M * D))

    @pl.when(kb > 0)
    def _():
        h_ref[0] = h_ref[0] + hblk


def _finalize_body(x_ref, xd0_ref, xd1_ref, c0_ref, h_ref, q_ref, cp_ref,
                   perp_ref):
    xf = x_ref[0]
    a = xd0_ref[0]
    b = xd1_ref[0]
    q_ref[0] = a + b
    e = (xf - a) - b
    c1 = jnp.sum((e * e).reshape(1, M * D), axis=1, keepdims=True) \
        * (1.0 / (M * D))
    cp_ref[0] = c0_ref[0] + c1
    perp_ref[0] = jnp.exp(-h_ref[0])


_x_spec = pl.BlockSpec((1, M, D), lambda g, kb: (g, 0, 0))
_cb_spec = pl.BlockSpec((1, KBLK, D), lambda g, kb: (g, kb, 0))
_idx_spec = pl.BlockSpec((1, 1, M), lambda g, kb: (g, 0, 0))
_scal_spec = pl.BlockSpec((1, 1, 1), lambda g, kb: (g, 0, 0))


def _run_layer1(xf, cb):
    return pl.pallas_call(
        _layer1_body,
        grid=(G, NKB),
        in_specs=[_x_spec, _cb_spec],
        out_specs=[_idx_spec],
        out_shape=[jax.ShapeDtypeStruct((G, 1, M), jnp.int32)],
        scratch_shapes=[pltpu.VMEM((1, M), jnp.float32)],
    )(xf, cb)[0]


def _run_layer2(xf, xd0, cb, idx0):
    return pl.pallas_call(
        _layer2_body,
        grid=(G, NKB),
        in_specs=[_x_spec, _x_spec, _cb_spec, _idx_spec],
        out_specs=[_idx_spec, _scal_spec, _scal_spec],
        out_shape=[jax.ShapeDtypeStruct((G, 1, M), jnp.int32),
                   jax.ShapeDtypeStruct((G, 1, 1), jnp.float32),
                   jax.ShapeDtypeStruct((G, 1, 1), jnp.float32)],
        scratch_shapes=[pltpu.VMEM((1, M), jnp.float32)],
    )(xf, xd0, cb, idx0)


def _run_finalize(xf, xd0, xd1, c0, h):
    s_x = pl.BlockSpec((1, M, D), lambda g: (g, 0, 0))
    s_s = pl.BlockSpec((1, 1, 1), lambda g: (g, 0, 0))
    return pl.pallas_call(
        _finalize_body,
        grid=(G,),
        in_specs=[s_x, s_x, s_x, s_s, s_s],
        out_specs=[s_x, s_s, s_s],
        out_shape=[jax.ShapeDtypeStruct((G, M, D), jnp.float32),
                   jax.ShapeDtypeStruct((G, 1, 1), jnp.float32),
                   jax.ShapeDtypeStruct((G, 1, 1), jnp.float32)],
    )(xf, xd0, xd1, c0, h)


_NC, _NS = 2, 16
_ROWS_PER_W = (G * M) // (_NC * _NS)


def _sc_gather(cb_flat, idx_flat):
    mesh = plsc.VectorSubcoreMesh(core_axis_name="c", subcore_axis_name="s")

    @functools.partial(
        pl.kernel, mesh=mesh,
        out_type=jax.ShapeDtypeStruct((G * M, D), jnp.float32),
        scratch_types=[
            pltpu.VMEM((_ROWS_PER_W,), jnp.int32),
            pltpu.VMEM((_ROWS_PER_W, D), jnp.float32),
            pltpu.SemaphoreType.DMA,
        ],
    )
    def k(cb_hbm, idx_hbm, out_hbm, idx_v, rows_v, sem):
        c = lax.axis_index("c")
        s = lax.axis_index("s")
        base = c * M + s * _ROWS_PER_W
        pltpu.sync_copy(idx_hbm.at[pl.ds(base, _ROWS_PER_W)], idx_v)
        off = c * K
        for j in range(_ROWS_PER_W // 16):
            sl = pl.ds(j * 16, 16)
            idx_v[sl] = idx_v[sl] + off
        pltpu.async_copy(cb_hbm.at[idx_v], rows_v, sem).wait()
        pltpu.sync_copy(rows_v, out_hbm.at[pl.ds(base, _ROWS_PER_W)])

    return k(cb_flat, idx_flat)


def kernel(x, codebooks):
    xt = jnp.transpose(x, (0, 2, 1)).reshape(M, G * D)
    xf = jnp.stack([xt[:, g * D:(g + 1) * D] for g in range(G)])
    cb0 = codebooks[:, 0]
    cb1 = codebooks[:, 1]

    idx0 = _run_layer1(xf, cb0)
    xd0 = _sc_gather(cb0.reshape(G * K, D),
                     idx0.reshape(G * M)).reshape(G, M, D)
    idx1, c0, h = _run_layer2(xf, xd0, cb1, idx0)
    xd1 = _sc_gather(cb1.reshape(G * K, D),
                     idx1.reshape(G * M)).reshape(G, M, D)
    q, cp, perp = _run_finalize(xf, xd0, xd1, c0, h)

    quantized = jnp.transpose(
        jnp.concatenate([q[g].reshape(4, 512, D) for g in range(G)], axis=2),
        (0, 2, 1))
    commit_total = cp[0, 0, 0] + cp[1, 0, 0]
    return quantized, commit_total, perp.reshape(G), idx0.reshape(G, M)


# speedup vs baseline: 1.6175x; 1.2763x over previous
import functools

import jax
import jax.numpy as jnp
from jax import lax
from jax.experimental import pallas as pl
from jax.experimental.pallas import tpu as pltpu
from jax.experimental.pallas import tpu_sc as plsc

G = 2
K = 8192
D = 256
M = 2048
KBLK = 1024
NKB = K // KBLK


def _layer_step(xf, cb, kb, idx_ref, accv, acckb):
    xn = jnp.sum(xf * xf, axis=1)
    cn = jnp.sum(cb * cb, axis=1)
    d = lax.dot_general(xf, cb, (((1,), (1,)), ((), ())),
                        preferred_element_type=jnp.float32)
    s = (xn[:, None] - 2.0 * d) + cn[None, :]

    @pl.when(kb == 0)
    def _():
        accv[...] = s
        acckb[...] = jnp.zeros((M, KBLK), jnp.int8)

    @pl.when(kb > 0)
    def _():
        prev = accv[...]
        upd = s < prev
        accv[...] = jnp.minimum(s, prev)
        acckb[...] = jnp.where(upd, jnp.full((M, KBLK), kb, jnp.int8),
                               acckb[...])

    @pl.when(kb == NKB - 1)
    def _():
        av = accv[...]
        m = jnp.min(av, axis=1)
        jcol = lax.broadcasted_iota(jnp.int32, (M, KBLK), 1)
        gidx = acckb[...].astype(jnp.int32) * KBLK + jcol
        cand = jnp.where(av == m[:, None], gidx, K)
        idx_ref[0] = jnp.min(cand, axis=1)[None]


def _layer1_body(x_ref, cb_ref, idx_ref, accv, acckb):
    _layer_step(x_ref[0], cb_ref[0], pl.program_id(1), idx_ref, accv, acckb)


def _layer2_body(x_ref, xd_ref, cb_ref, idx_ref, accv, acckb):
    _layer_step(x_ref[0] - xd_ref[0], cb_ref[0], pl.program_id(1), idx_ref,
                accv, acckb)


def _finalize_body(x_ref, xd0_ref, xd1_ref, cnt_ref, q_ref, cp_ref, perp_ref):
    xf = x_ref[0]
    a = xd0_ref[0]
    b = xd1_ref[0]
    q_ref[0] = a + b
    r0 = xf - a
    c0 = jnp.sum((r0 * r0).reshape(1, M * D), axis=1, keepdims=True)
    e = r0 - b
    c1 = jnp.sum((e * e).reshape(1, M * D), axis=1, keepdims=True)
    cp_ref[0] = (c0 + c1) * (1.0 / (M * D))
    prob = cnt_ref[0] * (1.0 / M)
    h = jnp.sum(prob * jnp.log(prob + 1e-07), axis=1, keepdims=True)
    perp_ref[0] = jnp.exp(-h)


_x_spec = pl.BlockSpec((1, M, D), lambda g, kb: (g, 0, 0))
_cb_spec = pl.BlockSpec((1, KBLK, D), lambda g, kb: (g, kb, 0))
_idx_spec = pl.BlockSpec((1, 1, M), lambda g, kb: (g, 0, 0))


def _run_layer1(xf, cb):
    return pl.pallas_call(
        _layer1_body,
        grid=(G, NKB),
        in_specs=[_x_spec, _cb_spec],
        out_specs=[_idx_spec],
        out_shape=[jax.ShapeDtypeStruct((G, 1, M), jnp.int32)],
        scratch_shapes=[pltpu.VMEM((M, KBLK), jnp.float32),
                        pltpu.VMEM((M, KBLK), jnp.int8)],
    )(xf, cb)[0]


def _run_layer2(xf, xd0, cb):
    return pl.pallas_call(
        _layer2_body,
        grid=(G, NKB),
        in_specs=[_x_spec, _x_spec, _cb_spec],
        out_specs=[_idx_spec],
        out_shape=[jax.ShapeDtypeStruct((G, 1, M), jnp.int32)],
        scratch_shapes=[pltpu.VMEM((M, KBLK), jnp.float32),
                        pltpu.VMEM((M, KBLK), jnp.int8)],
    )(xf, xd0, cb)[0]


def _run_finalize(xf, xd0, xd1, counts):
    s_x = pl.BlockSpec((1, M, D), lambda g: (g, 0, 0))
    s_s = pl.BlockSpec((1, 1, 1), lambda g: (g, 0, 0))
    s_c = pl.BlockSpec((1, 1, K), lambda g: (g, 0, 0))
    return pl.pallas_call(
        _finalize_body,
        grid=(G,),
        in_specs=[s_x, s_x, s_x, s_c],
        out_specs=[s_x, s_s, s_s],
        out_shape=[jax.ShapeDtypeStruct((G, M, D), jnp.float32),
                   jax.ShapeDtypeStruct((G, 1, 1), jnp.float32),
                   jax.ShapeDtypeStruct((G, 1, 1), jnp.float32)],
    )(xf, xd0, xd1, counts)


_NC, _NS = 2, 16
_RPW = (G * M) // (_NC * _NS)
_CPW = K // _NS


def _sc_gather(cb_flat, idx_flat, with_counts):
    mesh = plsc.VectorSubcoreMesh(core_axis_name="c", subcore_axis_name="s")
    out_type = [jax.ShapeDtypeStruct((G * M, D), jnp.float32)]
    if with_counts:
        out_type.append(jax.ShapeDtypeStruct((G * K,), jnp.float32))
    scratch = [
        pltpu.VMEM((_RPW,), jnp.int32),
        pltpu.VMEM((_RPW, D), jnp.float32),
        pltpu.SemaphoreType.DMA,
    ]
    if with_counts:
        scratch += [pltpu.VMEM((_CPW,), jnp.float32),
                    pltpu.VMEM((_RPW,), jnp.float32),
                    pltpu.VMEM_SHARED((K,), jnp.float32)]

    @functools.partial(pl.kernel, mesh=mesh, out_type=out_type,
                       scratch_types=scratch)
    def k(cb_hbm, idx_hbm, out_hbm, *rest):
        if with_counts:
            cnt_hbm, idx_v, rows_v, sem, slab_v, ones_v, shared = rest
        else:
            idx_v, rows_v, sem = rest
        c = lax.axis_index("c")
        s = lax.axis_index("s")
        base = c * M + s * _RPW
        pltpu.sync_copy(idx_hbm.at[pl.ds(base, _RPW)], idx_v)

        if with_counts:
            for j in range(_CPW // 16):
                slab_v[pl.ds(j * 16, 16)] = jnp.zeros((16,), jnp.float32)
            pltpu.sync_copy(slab_v, shared.at[pl.ds(s * _CPW, _CPW)])
            plsc.subcore_barrier()
            for j in range(_RPW // 16):
                ones_v[pl.ds(j * 16, 16)] = jnp.ones((16,), jnp.float32)
            pltpu.sync_copy(ones_v, shared.at[idx_v], add=True)
            plsc.subcore_barrier()
            pltpu.sync_copy(shared.at[pl.ds(s * _CPW, _CPW)], slab_v)
            pltpu.sync_copy(slab_v, cnt_hbm.at[pl.ds(c * K + s * _CPW, _CPW)])

        off = c * K
        for j in range(_RPW // 16):
            sl = pl.ds(j * 16, 16)
            idx_v[sl] = idx_v[sl] + off
        pltpu.async_copy(cb_hbm.at[idx_v], rows_v, sem).wait()
        pltpu.sync_copy(rows_v, out_hbm.at[pl.ds(base, _RPW)])

    return k(cb_flat, idx_flat)


def kernel(x, codebooks):
    xt = jnp.transpose(x, (0, 2, 1)).reshape(M, G * D)
    xf = jnp.stack([xt[:, g * D:(g + 1) * D] for g in range(G)])
    cb0 = codebooks[:, 0]
    cb1 = codebooks[:, 1]

    idx0 = _run_layer1(xf, cb0)
    xd0, counts = _sc_gather(cb0.reshape(G * K, D), idx0.reshape(G * M),
                             with_counts=True)
    xd0 = xd0.reshape(G, M, D)
    idx1 = _run_layer2(xf, xd0, cb1)
    xd1 = _sc_gather(cb1.reshape(G * K, D), idx1.reshape(G * M),
                     with_counts=False)[0].reshape(G, M, D)
    q, cp, perp = _run_finalize(xf, xd0, xd1, counts.reshape(G, 1, K))

    quantized = jnp.transpose(
        jnp.concatenate([q[g].reshape(4, 512, D) for g in range(G)], axis=2),
        (0, 2, 1))
    commit_total = cp[0, 0, 0] + cp[1, 0, 0]
    return quantized, commit_total, perp.reshape(G), idx0.reshape(G, M)


# speedup vs baseline: 1.6269x; 1.0058x over previous
import functools

import jax
import jax.numpy as jnp
from jax import lax
from jax.experimental import pallas as pl
from jax.experimental.pallas import tpu as pltpu
from jax.experimental.pallas import tpu_sc as plsc

G = 2
K = 8192
D = 256
M = 2048
KBLK = 2048
NKB = K // KBLK


def _layer_step(xf, cb, kb, idx_ref, accv, acckb):
    xn = jnp.sum(xf * xf, axis=1)
    cn = jnp.sum(cb * cb, axis=1)
    d = lax.dot_general(xf, cb, (((1,), (1,)), ((), ())),
                        preferred_element_type=jnp.float32)
    s = (xn[:, None] - 2.0 * d) + cn[None, :]

    @pl.when(kb == 0)
    def _():
        accv[...] = s
        acckb[...] = jnp.zeros((M, KBLK), jnp.int8)

    @pl.when(kb > 0)
    def _():
        prev = accv[...]
        upd = s < prev
        accv[...] = jnp.minimum(s, prev)
        acckb[...] = jnp.where(upd, jnp.full((M, KBLK), kb, jnp.int8),
                               acckb[...])

    @pl.when(kb == NKB - 1)
    def _():
        av = accv[...]
        m = jnp.min(av, axis=1)
        jcol = lax.broadcasted_iota(jnp.int32, (M, KBLK), 1)
        gidx = acckb[...].astype(jnp.int32) * KBLK + jcol
        cand = jnp.where(av == m[:, None], gidx, K)
        idx_ref[0] = jnp.min(cand, axis=1)[None]


def _layer1_body(x_ref, cb_ref, idx_ref, accv, acckb):
    _layer_step(x_ref[0], cb_ref[0], pl.program_id(1), idx_ref, accv, acckb)


def _layer2_body(x_ref, xd_ref, cb_ref, idx_ref, accv, acckb):
    _layer_step(x_ref[0] - xd_ref[0], cb_ref[0], pl.program_id(1), idx_ref,
                accv, acckb)


def _finalize_body(x_ref, xd0_ref, xd1_ref, cnt_ref, q_ref, cp_ref, perp_ref):
    xf = x_ref[0]
    a = xd0_ref[0]
    b = xd1_ref[0]
    q_ref[0] = a + b
    r0 = xf - a
    c0 = jnp.sum((r0 * r0).reshape(1, M * D), axis=1, keepdims=True)
    e = r0 - b
    c1 = jnp.sum((e * e).reshape(1, M * D), axis=1, keepdims=True)
    cp_ref[0] = (c0 + c1) * (1.0 / (M * D))
    prob = cnt_ref[0] * (1.0 / M)
    h = jnp.sum(prob * jnp.log(prob + 1e-07), axis=1, keepdims=True)
    perp_ref[0] = jnp.exp(-h)


_x_spec = pl.BlockSpec((1, M, D), lambda g, kb: (g, 0, 0))
_cb_spec = pl.BlockSpec((1, KBLK, D), lambda g, kb: (g, kb, 0))
_idx_spec = pl.BlockSpec((1, 1, M), lambda g, kb: (g, 0, 0))


def _run_layer1(xf, cb):
    return pl.pallas_call(
        _layer1_body,
        grid=(G, NKB),
        in_specs=[_x_spec, _cb_spec],
        out_specs=[_idx_spec],
        out_shape=[jax.ShapeDtypeStruct((G, 1, M), jnp.int32)],
        scratch_shapes=[pltpu.VMEM((M, KBLK), jnp.float32),
                        pltpu.VMEM((M, KBLK), jnp.int8)],
        compiler_params=pltpu.CompilerParams(
            vmem_limit_bytes=100 * 1024 * 1024),
    )(xf, cb)[0]


def _run_layer2(xf, xd0, cb):
    return pl.pallas_call(
        _layer2_body,
        grid=(G, NKB),
        in_specs=[_x_spec, _x_spec, _cb_spec],
        out_specs=[_idx_spec],
        out_shape=[jax.ShapeDtypeStruct((G, 1, M), jnp.int32)],
        scratch_shapes=[pltpu.VMEM((M, KBLK), jnp.float32),
                        pltpu.VMEM((M, KBLK), jnp.int8)],
        compiler_params=pltpu.CompilerParams(
            vmem_limit_bytes=100 * 1024 * 1024),
    )(xf, xd0, cb)[0]


def _run_finalize(xf, xd0, xd1, counts):
    s_x = pl.BlockSpec((1, M, D), lambda g: (g, 0, 0))
    s_s = pl.BlockSpec((1, 1, 1), lambda g: (g, 0, 0))
    s_c = pl.BlockSpec((1, 1, K), lambda g: (g, 0, 0))
    return pl.pallas_call(
        _finalize_body,
        grid=(G,),
        in_specs=[s_x, s_x, s_x, s_c],
        out_specs=[s_x, s_s, s_s],
        out_shape=[jax.ShapeDtypeStruct((G, M, D), jnp.float32),
                   jax.ShapeDtypeStruct((G, 1, 1), jnp.float32),
                   jax.ShapeDtypeStruct((G, 1, 1), jnp.float32)],
    )(xf, xd0, xd1, counts)


_NC, _NS = 2, 16
_RPW = (G * M) // (_NC * _NS)
_CPW = K // _NS


def _sc_gather(cb_flat, idx_flat, with_counts):
    mesh = plsc.VectorSubcoreMesh(core_axis_name="c", subcore_axis_name="s")
    out_type = [jax.ShapeDtypeStruct((G * M, D), jnp.float32)]
    if with_counts:
        out_type.append(jax.ShapeDtypeStruct((G * K,), jnp.float32))
    scratch = [
        pltpu.VMEM((_RPW,), jnp.int32),
        pltpu.VMEM((_RPW, D), jnp.float32),
        pltpu.SemaphoreType.DMA,
    ]
    if with_counts:
        scratch += [pltpu.VMEM((_CPW,), jnp.float32),
                    pltpu.VMEM((_RPW,), jnp.float32),
                    pltpu.VMEM_SHARED((K,), jnp.float32)]

    @functools.partial(pl.kernel, mesh=mesh, out_type=out_type,
                       scratch_types=scratch)
    def k(cb_hbm, idx_hbm, out_hbm, *rest):
        if with_counts:
            cnt_hbm, idx_v, rows_v, sem, slab_v, ones_v, shared = rest
        else:
            idx_v, rows_v, sem = rest
        c = lax.axis_index("c")
        s = lax.axis_index("s")
        base = c * M + s * _RPW
        pltpu.sync_copy(idx_hbm.at[pl.ds(base, _RPW)], idx_v)

        if with_counts:
            for j in range(_CPW // 16):
                slab_v[pl.ds(j * 16, 16)] = jnp.zeros((16,), jnp.float32)
            pltpu.sync_copy(slab_v, shared.at[pl.ds(s * _CPW, _CPW)])
            plsc.subcore_barrier()
            for j in range(_RPW // 16):
                ones_v[pl.ds(j * 16, 16)] = jnp.ones((16,), jnp.float32)
            pltpu.sync_copy(ones_v, shared.at[idx_v], add=True)
            plsc.subcore_barrier()
            pltpu.sync_copy(shared.at[pl.ds(s * _CPW, _CPW)], slab_v)
            pltpu.sync_copy(slab_v, cnt_hbm.at[pl.ds(c * K + s * _CPW, _CPW)])

        off = c * K
        for j in range(_RPW // 16):
            sl = pl.ds(j * 16, 16)
            idx_v[sl] = idx_v[sl] + off
        pltpu.async_copy(cb_hbm.at[idx_v], rows_v, sem).wait()
        pltpu.sync_copy(rows_v, out_hbm.at[pl.ds(base, _RPW)])

    return k(cb_flat, idx_flat)


def kernel(x, codebooks):
    xt = jnp.transpose(x, (0, 2, 1)).reshape(M, G * D)
    xf = jnp.stack([xt[:, g * D:(g + 1) * D] for g in range(G)])
    cb0 = codebooks[:, 0]
    cb1 = codebooks[:, 1]

    idx0 = _run_layer1(xf, cb0)
    xd0, counts = _sc_gather(cb0.reshape(G * K, D), idx0.reshape(G * M),
                             with_counts=True)
    xd0 = xd0.reshape(G, M, D)
    idx1 = _run_layer2(xf, xd0, cb1)
    xd1 = _sc_gather(cb1.reshape(G * K, D), idx1.reshape(G * M),
                     with_counts=False)[0].reshape(G, M, D)
    q, cp, perp = _run_finalize(xf, xd0, xd1, counts.reshape(G, 1, K))

    quantized = jnp.transpose(
        jnp.concatenate([q[g].reshape(4, 512, D) for g in range(G)], axis=2),
        (0, 2, 1))
    commit_total = cp[0, 0, 0] + cp[1, 0, 0]
    return quantized, commit_total, perp.reshape(G), idx0.reshape(G, M)
